# Initial kernel scaffold; baseline (speedup 1.0000x reference)
import jax
import jax.numpy as jnp
from jax.experimental import pallas as pl


def kernel(pos_s, pos_e, pe, W, b):
    raise NotImplementedError("write your pallas kernel here")


# speedup vs baseline: 6.3398x; 6.3398x over previous
import functools
import math

import jax
import jax.numpy as jnp
from jax import lax
from jax.experimental import pallas as pl
from jax.experimental.pallas import tpu as pltpu
from jax.experimental.pallas import tpu_sc as plsc

HIDDEN = 128
MAX_SEQ = 512
TBL_PAD = 1032

NUM_WORKERS = 32
WIN = 128


def _prep_kernel(pe_ref, w_ref, b_ref, ps_ref, pe2_ref,
                 t1_ref, t2_ref, iss_ref, iee_ref):
    w1 = w_ref[:HIDDEN, :]
    w2 = w_ref[HIDDEN:, :]
    pe = pe_ref[...]
    t1_ref[...] = jnp.dot(pe, w1, preferred_element_type=jnp.float32) + b_ref[...]
    t2_ref[...] = jnp.dot(pe, w2, preferred_element_type=jnp.float32)
    ps = ps_ref[...]
    pe2 = pe2_ref[...]
    iss_ref[...] = ps[:, :, None] - ps[:, None, :] + MAX_SEQ
    iee_ref[...] = pe2[:, :, None] - pe2[:, None, :] + MAX_SEQ


def _make_sc_kernel(n_rows):
    rows_per_worker = n_rows // NUM_WORKERS
    n_win = rows_per_worker // WIN
    mesh = plsc.VectorSubcoreMesh(core_axis_name="c", subcore_axis_name="s")

    @functools.partial(
        pl.kernel,
        mesh=mesh,
        out_type=jax.ShapeDtypeStruct((n_rows, HIDDEN), jnp.float32),
        scratch_types=[
            pltpu.VMEM((WIN,), jnp.int32),
            pltpu.VMEM((WIN,), jnp.int32),
            pltpu.VMEM((WIN, HIDDEN), jnp.float32),
            pltpu.VMEM((WIN, HIDDEN), jnp.float32),
            pltpu.VMEM((WIN, HIDDEN), jnp.float32),
            pltpu.SemaphoreType.DMA,
            pltpu.SemaphoreType.DMA,
        ],
    )
    def sc_fused(t1_hbm, t2_hbm, iss_hbm, iee_hbm, out_hbm,
                 ia_v, ib_v, g1_v, g2_v, o_v, sem1, sem2):
        wid = lax.axis_index("s") * 2 + lax.axis_index("c")

        @pl.loop(0, n_win)
        def _win(w):
            base = (wid * n_win + w) * WIN
            pltpu.sync_copy(iss_hbm.at[pl.ds(base, WIN)], ia_v)
            pltpu.sync_copy(iee_hbm.at[pl.ds(base, WIN)], ib_v)
            c1 = pltpu.async_copy(t1_hbm.at[ia_v], g1_v, sem1)
            c2 = pltpu.async_copy(t2_hbm.at[ib_v], g2_v, sem2)
            c1.wait()
            c2.wait()

            @pl.loop(0, WIN)
            def _row(r):
                for c in range(HIDDEN // 16):
                    s = pl.ds(c * 16, 16)
                    x = g1_v[r, s] + g2_v[r, s]
                    o_v[r, s] = jnp.maximum(x, x * jnp.float32(0.01))

            pltpu.sync_copy(o_v, out_hbm.at[pl.ds(base, WIN)])

    return sc_fused


def kernel(pos_s, pos_e, pe, W, b):
    batch, seq = pos_s.shape
    n_rows = batch * seq * seq

    pe_pad = jnp.pad(pe, ((0, TBL_PAD - pe.shape[0]), (0, 0)))
    b2 = b.reshape(1, HIDDEN)

    t1, t2, iss, iee = pl.pallas_call(
        _prep_kernel,
        out_shape=(
            jax.ShapeDtypeStruct((TBL_PAD, HIDDEN), jnp.float32),
            jax.ShapeDtypeStruct((TBL_PAD, HIDDEN), jnp.float32),
            jax.ShapeDtypeStruct((batch, seq, seq), jnp.int32),
            jax.ShapeDtypeStruct((batch, seq, seq), jnp.int32),
        ),
    )(pe_pad, W, b2, pos_s, pos_e)

    out = _make_sc_kernel(n_rows)(t1, t2, iss.reshape(-1), iee.reshape(-1))
    return out.reshape(batch, seq, seq, HIDDEN)


# speedup vs baseline: 8.4722x; 1.3364x over previous
import functools
import math

import jax
import jax.numpy as jnp
from jax import lax
from jax.experimental import pallas as pl
from jax.experimental.pallas import tpu as pltpu
from jax.experimental.pallas import tpu_sc as plsc

HIDDEN = 128
MAX_SEQ = 512
TBL_PAD = 1032

NUM_WORKERS = 32
WIN = 128
ROW_UNROLL = 4


def _prep_kernel(pe_ref, w_ref, b_ref, ps_ref, pe2_ref,
                 t1_ref, t2_ref, iss_ref, iee_ref):
    w1 = w_ref[:HIDDEN, :]
    w2 = w_ref[HIDDEN:, :]
    pe = pe_ref[...]
    t1_ref[...] = jnp.dot(pe, w1, preferred_element_type=jnp.float32) + b_ref[...]
    t2_ref[...] = jnp.dot(pe, w2, preferred_element_type=jnp.float32)
    ps = ps_ref[...]
    pe2 = pe2_ref[...]
    iss_ref[...] = ps[:, :, None] - ps[:, None, :] + MAX_SEQ
    iee_ref[...] = pe2[:, :, None] - pe2[:, None, :] + MAX_SEQ


def _make_sc_kernel(n_rows):
    rows_per_worker = n_rows // NUM_WORKERS
    n_win = rows_per_worker // WIN
    assert n_win % 2 == 0 and n_win >= 4
    mesh = plsc.VectorSubcoreMesh(core_axis_name="c", subcore_axis_name="s")

    @functools.partial(
        pl.kernel,
        mesh=mesh,
        out_type=jax.ShapeDtypeStruct((n_rows, HIDDEN), jnp.float32),
        scratch_types=(
            [pltpu.VMEM((WIN,), jnp.int32)] * 4
            + [pltpu.VMEM((WIN, HIDDEN), jnp.float32)] * 6
            + [pltpu.SemaphoreType.DMA] * 6
        ),
    )
    def sc_fused(t1_hbm, t2_hbm, iss_hbm, iee_hbm, out_hbm,
                 ia0, ia1, ib0, ib1, g10, g11, g20, g21, o0, o1,
                 si0, si1, sg0, sg1, so0, so1):
        wid = lax.axis_index("s") * 2 + lax.axis_index("c")
        IA, IB = (ia0, ia1), (ib0, ib1)
        G1, G2, O = (g10, g11), (g20, g21), (o0, o1)
        SI, SG, SO = (si0, si1), (sg0, sg1), (so0, so1)

        def base_of(w):
            return (wid * n_win + w) * WIN

        def fire_idx(w, b):
            base = base_of(w)
            pltpu.async_copy(iss_hbm.at[pl.ds(base, WIN)], IA[b], SI[b])
            pltpu.async_copy(iee_hbm.at[pl.ds(base, WIN)], IB[b], SI[b])

        def wait_idx(w, b):
            base = base_of(w)
            pltpu.make_async_copy(iss_hbm.at[pl.ds(base, WIN)], IA[b], SI[b]).wait()
            pltpu.make_async_copy(iee_hbm.at[pl.ds(base, WIN)], IB[b], SI[b]).wait()

        def fire_gather(b):
            pltpu.async_copy(t1_hbm.at[IA[b]], G1[b], SG[b])
            pltpu.async_copy(t2_hbm.at[IB[b]], G2[b], SG[b])

        def wait_gather(b):
            pltpu.make_async_copy(t1_hbm.at[IA[b]], G1[b], SG[b]).wait()
            pltpu.make_async_copy(t2_hbm.at[IB[b]], G2[b], SG[b]).wait()

        def fire_out(w, b):
            pltpu.async_copy(O[b], out_hbm.at[pl.ds(base_of(w), WIN)], SO[b])

        def wait_out(w, b):
            pltpu.make_async_copy(O[b], out_hbm.at[pl.ds(base_of(w), WIN)],
                                  SO[b]).wait()

        def compute(b):
            g1_v, g2_v, o_v = G1[b], G2[b], O[b]

            @pl.loop(0, WIN, step=ROW_UNROLL)
            def _row(r0):
                for dr in range(ROW_UNROLL):
                    r = r0 + dr
                    for c in range(HIDDEN // 16):
                        s = pl.ds(c * 16, 16)
                        x = g1_v[r, s] + g2_v[r, s]
                        o_v[r, s] = jnp.maximum(x, x * jnp.float32(0.01))

        fire_idx(0, 0)
        wait_idx(0, 0)
        fire_gather(0)
        fire_idx(1, 1)

        @pl.loop(0, n_win - 2, step=2)
        def _steady(w):
            wait_gather(0)
            wait_idx(w + 1, 1)
            fire_gather(1)
            fire_idx(w + 2, 0)

            @pl.when(w >= 2)
            def _():
                wait_out(w - 2, 0)

            compute(0)
            fire_out(w, 0)

            wait_gather(1)
            wait_idx(w + 2, 0)
            fire_gather(0)
            fire_idx(w + 3, 1)

            @pl.when(w >= 2)
            def _():
                wait_out(w - 1, 1)

            compute(1)
            fire_out(w + 1, 1)

        wait_gather(0)
        wait_idx(n_win - 1, 1)
        fire_gather(1)
        wait_out(n_win - 4, 0)
        compute(0)
        fire_out(n_win - 2, 0)

        wait_gather(1)
        wait_out(n_win - 3, 1)
        compute(1)
        fire_out(n_win - 1, 1)

        wait_out(n_win - 2, 0)
        wait_out(n_win - 1, 1)

    return sc_fused


def kernel(pos_s, pos_e, pe, W, b):
    batch, seq = pos_s.shape
    n_rows = batch * seq * seq

    pe_pad = jnp.pad(pe, ((0, TBL_PAD - pe.shape[0]), (0, 0)))
    b2 = b.reshape(1, HIDDEN)

    t1, t2, iss, iee = pl.pallas_call(
        _prep_kernel,
        out_shape=(
            jax.ShapeDtypeStruct((TBL_PAD, HIDDEN), jnp.float32),
            jax.ShapeDtypeStruct((TBL_PAD, HIDDEN), jnp.float32),
            jax.ShapeDtypeStruct((batch, seq, seq), jnp.int32),
            jax.ShapeDtypeStruct((batch, seq, seq), jnp.int32),
        ),
    )(pe_pad, W, b2, pos_s, pos_e)

    out = _make_sc_kernel(n_rows)(t1, t2, iss.reshape(-1), iee.reshape(-1))
    return out.reshape(batch, seq, seq, HIDDEN)


# speedup vs baseline: 15.8308x; 1.8686x over previous
import functools
import math

import jax
import jax.numpy as jnp
from jax import lax
from jax.experimental import pallas as pl
from jax.experimental.pallas import tpu as pltpu
from jax.experimental.pallas import tpu_sc as plsc

HIDDEN = 128
MAX_SEQ = 512
TBL_PAD = 1032

NUM_WORKERS = 32
WIN = 128
ROW_UNROLL = 4


def _prep_kernel(pe_ref, w_ref, b_ref, ps_ref, pe2_ref,
                 t1_ref, t2_ref, iss_ref, iee_ref):
    w1 = w_ref[:HIDDEN, :]
    w2 = w_ref[HIDDEN:, :]
    pe = pe_ref[...]
    t1_ref[...] = jnp.dot(pe, w1, preferred_element_type=jnp.float32) + b_ref[...]
    t2_ref[...] = jnp.dot(pe, w2, preferred_element_type=jnp.float32)
    ps = ps_ref[...]
    pe2 = pe2_ref[...]
    iss_ref[...] = ps[:, :, None] - ps[:, None, :] + MAX_SEQ
    iee_ref[...] = pe2[:, :, None] - pe2[:, None, :] + MAX_SEQ


def _make_sc_kernel(n_rows):
    rows_per_worker = n_rows // NUM_WORKERS
    n_win = rows_per_worker // WIN
    assert n_win % 2 == 0 and n_win >= 4
    mesh = plsc.VectorSubcoreMesh(core_axis_name="c", subcore_axis_name="s")

    @functools.partial(
        pl.kernel,
        mesh=mesh,
        out_type=jax.ShapeDtypeStruct((n_rows, HIDDEN), jnp.float32),
        scratch_types=(
            [pltpu.VMEM((WIN,), jnp.int32)] * 4
            + [pltpu.VMEM((WIN, HIDDEN), jnp.float32)] * 6
            + [pltpu.VMEM_SHARED((TBL_PAD, HIDDEN), jnp.float32)] * 2
            + [pltpu.SemaphoreType.DMA] * 6
        ),
    )
    def sc_fused(t1_hbm, t2_hbm, iss_hbm, iee_hbm, out_hbm,
                 ia0, ia1, ib0, ib1, g10, g11, g20, g21, o0, o1,
                 t1_sh, t2_sh,
                 si0, si1, sg0, sg1, so0, so1):
        sid = lax.axis_index("s")
        wid = sid * 2 + lax.axis_index("c")

        @pl.when(sid == 0)
        def _():
            pltpu.sync_copy(t1_hbm, t1_sh)
            pltpu.sync_copy(t2_hbm, t2_sh)

        plsc.subcore_barrier()
        IA, IB = (ia0, ia1), (ib0, ib1)
        G1, G2, O = (g10, g11), (g20, g21), (o0, o1)
        SI, SG, SO = (si0, si1), (sg0, sg1), (so0, so1)

        def base_of(w):
            return (wid * n_win + w) * WIN

        def fire_idx(w, b):
            base = base_of(w)
            pltpu.async_copy(iss_hbm.at[pl.ds(base, WIN)], IA[b], SI[b])
            pltpu.async_copy(iee_hbm.at[pl.ds(base, WIN)], IB[b], SI[b])

        def wait_idx(w, b):
            base = base_of(w)
            pltpu.make_async_copy(iss_hbm.at[pl.ds(base, WIN)], IA[b], SI[b]).wait()
            pltpu.make_async_copy(iee_hbm.at[pl.ds(base, WIN)], IB[b], SI[b]).wait()

        def fire_gather(b):
            pltpu.async_copy(t1_sh.at[IA[b]], G1[b], SG[b])
            pltpu.async_copy(t2_sh.at[IB[b]], G2[b], SG[b])

        def wait_gather(b):
            pltpu.make_async_copy(t1_sh.at[IA[b]], G1[b], SG[b]).wait()
            pltpu.make_async_copy(t2_sh.at[IB[b]], G2[b], SG[b]).wait()

        def fire_out(w, b):
            pltpu.async_copy(O[b], out_hbm.at[pl.ds(base_of(w), WIN)], SO[b])

        def wait_out(w, b):
            pltpu.make_async_copy(O[b], out_hbm.at[pl.ds(base_of(w), WIN)],
                                  SO[b]).wait()

        def compute(b):
            g1_v, g2_v, o_v = G1[b], G2[b], O[b]

            @pl.loop(0, WIN, step=ROW_UNROLL)
            def _row(r0):
                for dr in range(ROW_UNROLL):
                    r = r0 + dr
                    for c in range(HIDDEN // 16):
                        s = pl.ds(c * 16, 16)
                        x = g1_v[r, s] + g2_v[r, s]
                        o_v[r, s] = jnp.maximum(x, x * jnp.float32(0.01))

        fire_idx(0, 0)
        wait_idx(0, 0)
        fire_gather(0)
        fire_idx(1, 1)

        @pl.loop(0, n_win - 2, step=2)
        def _steady(w):
            wait_gather(0)
            wait_idx(w + 1, 1)
            fire_gather(1)
            fire_idx(w + 2, 0)

            @pl.when(w >= 2)
            def _():
                wait_out(w - 2, 0)

            compute(0)
            fire_out(w, 0)

            wait_gather(1)
            wait_idx(w + 2, 0)
            fire_gather(0)
            fire_idx(w + 3, 1)

            @pl.when(w >= 2)
            def _():
                wait_out(w - 1, 1)

            compute(1)
            fire_out(w + 1, 1)

        wait_gather(0)
        wait_idx(n_win - 1, 1)
        fire_gather(1)
        wait_out(n_win - 4, 0)
        compute(0)
        fire_out(n_win - 2, 0)

        wait_gather(1)
        wait_out(n_win - 3, 1)
        compute(1)
        fire_out(n_win - 1, 1)

        wait_out(n_win - 2, 0)
        wait_out(n_win - 1, 1)

    return sc_fused


def kernel(pos_s, pos_e, pe, W, b):
    batch, seq = pos_s.shape
    n_rows = batch * seq * seq

    pe_pad = jnp.pad(pe, ((0, TBL_PAD - pe.shape[0]), (0, 0)))
    b2 = b.reshape(1, HIDDEN)

    t1, t2, iss, iee = pl.pallas_call(
        _prep_kernel,
        out_shape=(
            jax.ShapeDtypeStruct((TBL_PAD, HIDDEN), jnp.float32),
            jax.ShapeDtypeStruct((TBL_PAD, HIDDEN), jnp.float32),
            jax.ShapeDtypeStruct((batch, seq, seq), jnp.int32),
            jax.ShapeDtypeStruct((batch, seq, seq), jnp.int32),
        ),
    )(pe_pad, W, b2, pos_s, pos_e)

    out = _make_sc_kernel(n_rows)(t1, t2, iss.reshape(-1), iee.reshape(-1))
    return out.reshape(batch, seq, seq, HIDDEN)
